# baseline (speedup 1.0000x reference)
import functools

import jax
import jax.numpy as jnp
from jax import lax
from jax.experimental import pallas as pl
from jax.experimental.pallas import tpu as pltpu
from jax.experimental.pallas import tpu_sc as plsc

E = 16
D = 1024
NTOK = 4096
BLK = 128
P = NTOK + E * BLK
PB = P // BLK

NC, NS, L = 2, 16, 16
NW = NC * NS

_MESH = plsc.VectorSubcoreMesh(
    core_axis_name="c", subcore_axis_name="s", num_cores=NC, num_subcores=NS)


def _wid():
  return lax.axis_index("s") * NC + lax.axis_index("c")


TOK_T = NTOK // NS


def _routing_body(gate_hbm, slots_hbm, tfs_hbm, bexp_hbm,
                  gate_v, slots2, ramp2, cur_v, tmp_v, hist_v, bexp_v,
                  zero_v, hist_sh, tfs_sh, sem):
  t = lax.axis_index("s")

  @pl.when(lax.axis_index("c") == 0)
  def _():
    pltpu.sync_copy(gate_hbm.at[pl.ds(t * TOK_T, TOK_T)], gate_v)
    iota = lax.iota(jnp.int32, L)

    def chunk(i, assign):
      g = gate_v[pl.ds(i * L, L)]
      s, v = plsc.sort_key_val(g, iota + i * L)
      tmp_v[...] = s
      sp = plsc.load_gather(tmp_v, [jnp.maximum(iota - 1, 0)])
      sn = plsc.load_gather(tmp_v, [jnp.minimum(iota + 1, L - 1)])
      boundary = (iota == 0) | (s != sp)
      last = (iota == L - 1) | (s != sn)
      start = plsc.cummax(jnp.where(boundary, iota, 0))
      occ = iota - start
      curg = plsc.load_gather(cur_v, [s])
      slot = curg + occ
      if assign:
        plsc.store_scatter(slots2, [v // (TOK_T // 2), v % (TOK_T // 2)],
                           slot)
      plsc.store_scatter(cur_v, [s], slot + 1, mask=last)

    cur_v[...] = jnp.zeros((L,), jnp.int32)

    def pass1(i, c):
      chunk(i, False)
      return c
    lax.fori_loop(0, TOK_T // L, pass1, 0)

    pltpu.sync_copy(cur_v, hist_sh.at[pl.ds(t * L, L)])

    def zero(k, c):
      zero_v[pl.ds(k * L, L)] = jnp.zeros((L,), jnp.int32)
      return c
    lax.fori_loop(0, (P // NS) // L, zero, 0)
    pltpu.sync_copy(zero_v, tfs_sh.at[pl.ds(t * (P // NS), P // NS)])

    plsc.subcore_barrier()
    pltpu.sync_copy(hist_sh, hist_v)

    def acc_hist(tt, carry):
      cnt, pref = carry
      h = hist_v[pl.ds(tt * L, L)]
      return cnt + h, pref + jnp.where(tt < t, h, 0)
    cnt, pref = lax.fori_loop(
        0, NS, acc_hist,
        (jnp.zeros((L,), jnp.int32), jnp.zeros((L,), jnp.int32)))

    padded = ((cnt + BLK - 1) // BLK) * BLK
    inc = plsc.cumsum(padded)
    cur_v[...] = (inc - padded) + pref

    @pl.when(t == 0)
    def _():
      for vb in range(PB // L):
        mb = (iota + vb * L) * BLK
        acc = jnp.zeros((L,), jnp.int32)
        for e in range(E):
          end_e = jnp.sum(jnp.where(iota == e, inc, 0))
          acc = acc + jnp.where(mb >= end_e, 1, 0)
        bexp_v[pl.ds(vb * L, L)] = jnp.minimum(acc, E - 1)
      pltpu.sync_copy(bexp_v, bexp_hbm)

    for j in range(2):
      for k in range(TOK_T // 2 // L):
        ramp2[j, pl.ds(k * L, L)] = t * TOK_T + j * (TOK_T // 2) + k * L + iota

    def pass2(i, c):
      chunk(i, True)
      return c
    lax.fori_loop(0, TOK_T // L, pass2, 0)

    for j in range(2):
      pltpu.sync_copy(
          slots2.at[j],
          slots_hbm.at[pl.ds(t * TOK_T + j * (TOK_T // 2), TOK_T // 2)])
      pltpu.sync_copy(ramp2.at[j], tfs_sh.at[slots2.at[j]], add=True)

    plsc.subcore_barrier()
    pltpu.sync_copy(tfs_sh.at[pl.ds(t * (P // NS), P // NS)],
                    tfs_hbm.at[pl.ds(t * (P // NS), P // NS)])


_routing = pl.kernel(
    _routing_body,
    out_type=(
        jax.ShapeDtypeStruct((NTOK,), jnp.int32),
        jax.ShapeDtypeStruct((P,), jnp.int32),
        jax.ShapeDtypeStruct((PB,), jnp.int32),
    ),
    mesh=_MESH,
    compiler_params=pltpu.CompilerParams(needs_layout_passes=False),
    scratch_types=[
        pltpu.VMEM((TOK_T,), jnp.int32),
        pltpu.VMEM((2, TOK_T // 2), jnp.int32),
        pltpu.VMEM((2, TOK_T // 2), jnp.int32),
        pltpu.VMEM((L,), jnp.int32),
        pltpu.VMEM((L,), jnp.int32),
        pltpu.VMEM((NS * L,), jnp.int32),
        pltpu.VMEM((PB,), jnp.int32),
        pltpu.VMEM((P // NS,), jnp.int32),
        pltpu.VMEM_SHARED((NS * L,), jnp.int32),
        pltpu.VMEM_SHARED((P,), jnp.int32),
        pltpu.SemaphoreType.DMA,
    ],
)


ROWS_W = P // NW
GCH = 64


def _gather_body(tfs_hbm, inp_hbm, xbuf_hbm, idx_v, rows_v, sem):
  base = _wid() * ROWS_W

  pltpu.sync_copy(tfs_hbm.at[pl.ds(base, ROWS_W)], idx_v)

  def clamp(k, carry):
    v = idx_v[pl.ds(k * L, L)]
    idx_v[pl.ds(k * L, L)] = jnp.clip(v, 0, NTOK - 1)
    return carry
  lax.fori_loop(0, ROWS_W // L, clamp, 0)

  def step(c, carry):
    pltpu.async_copy(inp_hbm.at[idx_v.at[pl.ds(c * GCH, GCH)]], rows_v,
                     sem).wait()
    pltpu.sync_copy(rows_v, xbuf_hbm.at[pl.ds(base + c * GCH, GCH)])
    return carry
  lax.fori_loop(0, ROWS_W // GCH, step, 0)


_gather = pl.kernel(
    _gather_body,
    out_type=jax.ShapeDtypeStruct((P, D), jnp.float32),
    mesh=_MESH,
    scratch_types=[
        pltpu.VMEM((ROWS_W,), jnp.int32),
        pltpu.VMEM((GCH, D), jnp.float32),
        pltpu.SemaphoreType.DMA,
    ],
)


def _gemm_body(bexp_ref, x_ref, w_ref, o_ref):
  o_ref[...] = lax.dot_general(
      x_ref[...], w_ref[0],
      dimension_numbers=(((1,), (1,)), ((), ())),
      preferred_element_type=jnp.float32)


_gemm = pl.pallas_call(
    _gemm_body,
    grid_spec=pltpu.PrefetchScalarGridSpec(
        num_scalar_prefetch=1,
        grid=(PB,),
        in_specs=[
            pl.BlockSpec((BLK, D), lambda i, bexp: (i, 0)),
            pl.BlockSpec((1, D, D), lambda i, bexp: (bexp[i], 0, 0)),
        ],
        out_specs=pl.BlockSpec((BLK, D), lambda i, bexp: (i, 0)),
    ),
    out_shape=jax.ShapeDtypeStruct((P, D), jnp.float32),
)


TOK_W = NTOK // NW
BCH = 64


def _back_body(slots_hbm, ybuf_hbm, out_hbm, idx_v, rows_v, sem):
  base = _wid() * TOK_W

  pltpu.sync_copy(slots_hbm.at[pl.ds(base, TOK_W)], idx_v)

  def step(c, carry):
    pltpu.async_copy(ybuf_hbm.at[idx_v.at[pl.ds(c * BCH, BCH)]], rows_v,
                     sem).wait()
    pltpu.sync_copy(rows_v, out_hbm.at[pl.ds(base + c * BCH, BCH)])
    return carry
  lax.fori_loop(0, TOK_W // BCH, step, 0)


_back = pl.kernel(
    _back_body,
    out_type=jax.ShapeDtypeStruct((NTOK, D), jnp.float32),
    mesh=_MESH,
    scratch_types=[
        pltpu.VMEM((TOK_W,), jnp.int32),
        pltpu.VMEM((BCH, D), jnp.float32),
        pltpu.SemaphoreType.DMA,
    ],
)


@jax.jit
def kernel(inp, gate, weight):
  slots, tfs, bexp = _routing(gate)
  x_buf = _gather(tfs, inp)
  y_buf = _gemm(bexp, x_buf, weight)
  return _back(slots, y_buf)


# speedup vs baseline: 1.9015x; 1.9015x over previous
import functools

import jax
import jax.numpy as jnp
from jax import lax
from jax.experimental import pallas as pl
from jax.experimental.pallas import tpu as pltpu
from jax.experimental.pallas import tpu_sc as plsc

E = 16
D = 1024
NTOK = 4096
BLK = 128
P = NTOK + E * BLK
PB = P // BLK

NC, NS, L = 2, 16, 16
NW = NC * NS

_MESH = plsc.VectorSubcoreMesh(
    core_axis_name="c", subcore_axis_name="s", num_cores=NC, num_subcores=NS)


def _wid():
  return lax.axis_index("s") * NC + lax.axis_index("c")


TOK_T = NTOK // NS
RCH = 64


def _routing_body(gate_hbm, inp_hbm, slots_hbm, bexp_hbm, xbuf_hbm,
                  gate_v, slots4, cur_v, tmp_v, hist_v, bexp_v, rows_v,
                  hist_sh, sem):
  t = lax.axis_index("s")
  c = lax.axis_index("c")
  iota = lax.iota(jnp.int32, L)

  pltpu.sync_copy(gate_hbm.at[pl.ds(t * TOK_T, TOK_T)], gate_v)

  def chunk(i, assign):
    g = gate_v[pl.ds(i * L, L)]
    s, v = plsc.sort_key_val(g, iota + i * L)
    tmp_v[...] = s
    sp = plsc.load_gather(tmp_v, [jnp.maximum(iota - 1, 0)])
    sn = plsc.load_gather(tmp_v, [jnp.minimum(iota + 1, L - 1)])
    boundary = (iota == 0) | (s != sp)
    last = (iota == L - 1) | (s != sn)
    start = plsc.cummax(jnp.where(boundary, iota, 0))
    occ = iota - start
    curg = plsc.load_gather(cur_v, [s])
    slot = curg + occ
    if assign:
      plsc.store_scatter(slots4, [v // RCH, v % RCH], slot)
    plsc.store_scatter(cur_v, [s], slot + 1, mask=last)

  cur_v[...] = jnp.zeros((L,), jnp.int32)

  def pass1(i, carry):
    chunk(i, False)
    return carry
  lax.fori_loop(0, TOK_T // L, pass1, 0)

  pltpu.sync_copy(cur_v, hist_sh.at[pl.ds(t * L, L)])
  plsc.subcore_barrier()
  pltpu.sync_copy(hist_sh, hist_v)

  def acc_hist(tt, carry):
    cnt, pref = carry
    h = hist_v[pl.ds(tt * L, L)]
    return cnt + h, pref + jnp.where(tt < t, h, 0)
  cnt, pref = lax.fori_loop(
      0, NS, acc_hist,
      (jnp.zeros((L,), jnp.int32), jnp.zeros((L,), jnp.int32)))

  padded = ((cnt + BLK - 1) // BLK) * BLK
  inc = plsc.cumsum(padded)
  cur_v[...] = (inc - padded) + pref

  @pl.when((t == 0) & (c == 0))
  def _():
    for vb in range(PB // L):
      mb = (iota + vb * L) * BLK
      acc = jnp.zeros((L,), jnp.int32)
      for e in range(E):
        end_e = jnp.sum(jnp.where(iota == e, inc, 0))
        acc = acc + jnp.where(mb >= end_e, 1, 0)
      bexp_v[pl.ds(vb * L, L)] = jnp.minimum(acc, E - 1)
    pltpu.sync_copy(bexp_v, bexp_hbm)

  def pass2(i, carry):
    chunk(i, True)
    return carry
  lax.fori_loop(0, TOK_T // L, pass2, 0)

  @pl.when(c == 0)
  def _():
    for j in range(TOK_T // RCH):
      pltpu.sync_copy(slots4.at[j],
                      slots_hbm.at[pl.ds(t * TOK_T + j * RCH, RCH)])

  def dispatch(sc):
    def _():
      for h in range(TOK_T // RCH // NC):
        r = sc * (TOK_T // RCH // NC) + h
        pltpu.sync_copy(inp_hbm.at[pl.ds(t * TOK_T + r * RCH, RCH)], rows_v)
        pltpu.async_copy(rows_v, xbuf_hbm.at[slots4.at[r]], sem).wait()
    return _

  for sc in range(NC):
    pl.when(c == sc)(dispatch(sc))


_routing = pl.kernel(
    _routing_body,
    out_type=(
        jax.ShapeDtypeStruct((NTOK,), jnp.int32),
        jax.ShapeDtypeStruct((PB,), jnp.int32),
        jax.ShapeDtypeStruct((P, D), jnp.float32),
    ),
    mesh=_MESH,
    compiler_params=pltpu.CompilerParams(needs_layout_passes=False),
    scratch_types=[
        pltpu.VMEM((TOK_T,), jnp.int32),
        pltpu.VMEM((TOK_T // RCH, RCH), jnp.int32),
        pltpu.VMEM((L,), jnp.int32),
        pltpu.VMEM((L,), jnp.int32),
        pltpu.VMEM((NS * L,), jnp.int32),
        pltpu.VMEM((PB,), jnp.int32),
        pltpu.VMEM((RCH, D), jnp.float32),
        pltpu.VMEM_SHARED((NS * L,), jnp.int32),
        pltpu.SemaphoreType.DMA,
    ],
)


def _gemm_body(bexp_ref, x_ref, w_ref, o_ref):
  o_ref[...] = lax.dot_general(
      x_ref[...], w_ref[0],
      dimension_numbers=(((1,), (1,)), ((), ())),
      preferred_element_type=jnp.float32)


_gemm = pl.pallas_call(
    _gemm_body,
    grid_spec=pltpu.PrefetchScalarGridSpec(
        num_scalar_prefetch=1,
        grid=(PB,),
        in_specs=[
            pl.BlockSpec((BLK, D), lambda i, bexp: (i, 0)),
            pl.BlockSpec((1, D, D), lambda i, bexp: (bexp[i], 0, 0)),
        ],
        out_specs=pl.BlockSpec((BLK, D), lambda i, bexp: (i, 0)),
    ),
    out_shape=jax.ShapeDtypeStruct((P, D), jnp.float32),
)


TOK_W = NTOK // NW
BCH = 64


def _back_body(slots_hbm, ybuf_hbm, out_hbm, idx_v, rows_v, sem):
  base = _wid() * TOK_W

  pltpu.sync_copy(slots_hbm.at[pl.ds(base, TOK_W)], idx_v)

  def step(c, carry):
    pltpu.async_copy(ybuf_hbm.at[idx_v.at[pl.ds(c * BCH, BCH)]], rows_v,
                     sem).wait()
    pltpu.sync_copy(rows_v, out_hbm.at[pl.ds(base + c * BCH, BCH)])
    return carry
  lax.fori_loop(0, TOK_W // BCH, step, 0)


_back = pl.kernel(
    _back_body,
    out_type=jax.ShapeDtypeStruct((NTOK, D), jnp.float32),
    mesh=_MESH,
    scratch_types=[
        pltpu.VMEM((TOK_W,), jnp.int32),
        pltpu.VMEM((BCH, D), jnp.float32),
        pltpu.SemaphoreType.DMA,
    ],
)


@jax.jit
def kernel(inp, gate, weight):
  slots, bexp, x_buf = _routing(gate, inp)
  y_buf = _gemm(bexp, x_buf, weight)
  return _back(slots, y_buf)
